# baseline (speedup 1.0000x reference)
import functools

import jax
import jax.numpy as jnp
from jax import lax
from jax.experimental import pallas as pl
from jax.experimental.pallas import tpu as pltpu
from jax.experimental.pallas import tpu_sc as plsc

_NBUF = 2


@functools.lru_cache(maxsize=None)
def _build(num_rows, dim, b, s):
    info = plsc.get_sparse_core_info()
    nc, ns, nl = info.num_cores, info.num_subcores, info.num_lanes
    nw = nc * ns
    bw = b // nw
    assert b % nw == 0 and bw % nl == 0 and dim % nl == 0

    mesh = plsc.VectorSubcoreMesh(core_axis_name="c", subcore_axis_name="s")

    @functools.partial(
        pl.kernel,
        mesh=mesh,
        compiler_params=pltpu.CompilerParams(
            use_tc_tiling_on_sc=False, needs_layout_passes=False
        ),
        out_type=jax.ShapeDtypeStruct((s, dim, b), jnp.float32),
        scratch_types=[
            pltpu.VMEM((s, bw), jnp.int32),
            pltpu.VMEM((_NBUF, bw, dim), jnp.float32),
            pltpu.VMEM((_NBUF, dim, bw + 1), jnp.float32),
            pltpu.SemaphoreType.DMA,
            pltpu.SemaphoreType.DMA,
        ],
    )
    def gather_kernel(table, tab_t, idxs, out, idx_v, rows_v, trans_v, gsem, wsem):
        wid = lax.axis_index("s") * nc + lax.axis_index("c")
        base = wid * bw
        pltpu.sync_copy(idxs.at[:, pl.ds(base, bw)], idx_v)

        iota = lax.iota(jnp.int32, nl)

        def start_gather(c):
            return pltpu.async_copy(
                table.at[idx_v.at[c]], rows_v.at[lax.rem(c, _NBUF)], gsem
            )

        start_gather(0)

        def loop_body(c, carry):
            cm = lax.rem(c, _NBUF)

            @pl.when(c + 1 < s)
            def _():
                start_gather(c + 1)

            pltpu.make_async_copy(
                table.at[idx_v.at[c]], rows_v.at[cm], gsem
            ).wait()

            @pl.when(c >= _NBUF)
            def _():
                pltpu.make_async_copy(
                    trans_v.at[cm, :, pl.ds(0, bw)],
                    out.at[c - _NBUF, :, pl.ds(base, bw)],
                    wsem,
                ).wait()

            rows = rows_v.at[cm]
            trans = trans_v.at[cm]

            row_ids = [iota + h * nl for h in range(dim // nl)]
            for t in range(bw):
                col = jnp.full((nl,), t, jnp.int32)
                for h in range(dim // nl):
                    v = rows[t, pl.ds(h * nl, nl)]
                    plsc.store_scatter(trans, [row_ids[h], col], v)

            pltpu.async_copy(
                trans_v.at[cm, :, pl.ds(0, bw)],
                out.at[c, :, pl.ds(base, bw)],
                wsem,
            )
            return carry

        lax.fori_loop(0, s, loop_body, 0)
        for k in range(_NBUF):
            c = s - _NBUF + k
            pltpu.make_async_copy(
                trans_v.at[c % _NBUF, :, pl.ds(0, bw)],
                out.at[c, :, pl.ds(base, bw)],
                wsem,
            ).wait()

    return gather_kernel


def kernel(token_ids, embeddings):
    b, s = token_ids.shape
    v, d = embeddings.shape
    gather_kernel = _build(v, d, b, s)
    out_sdb = gather_kernel(embeddings, embeddings.T, token_ids.T)
    return jnp.transpose(out_sdb, (2, 0, 1))


# speedup vs baseline: 3.8359x; 3.8359x over previous
import functools

import jax
import jax.numpy as jnp
from jax import lax
from jax.experimental import pallas as pl
from jax.experimental.pallas import tpu as pltpu
from jax.experimental.pallas import tpu_sc as plsc

_NBUF = 3


@functools.lru_cache(maxsize=None)
def _build(num_rows, dim, b, s):
    info = plsc.get_sparse_core_info()
    nc, ns, nl = info.num_cores, info.num_subcores, info.num_lanes
    nw = nc * ns
    bw = b // nw
    assert b % nw == 0 and bw % nl == 0 and dim % nl == 0

    mesh = plsc.VectorSubcoreMesh(core_axis_name="c", subcore_axis_name="s")

    @functools.partial(
        pl.kernel,
        mesh=mesh,
        compiler_params=pltpu.CompilerParams(
            use_tc_tiling_on_sc=False, needs_layout_passes=False
        ),
        out_type=jax.ShapeDtypeStruct((s, dim, b), jnp.float32),
        scratch_types=[
            pltpu.VMEM((s, bw), jnp.int32),
            pltpu.VMEM((_NBUF, bw, dim), jnp.float32),
            pltpu.VMEM((_NBUF, dim, bw + 1), jnp.float32),
            pltpu.SemaphoreType.DMA,
            pltpu.SemaphoreType.DMA,
        ],
    )
    def gather_kernel(table, idxs, out, idx_v, rows_v, trans_v, gsem, wsem):
        wid = lax.axis_index("s") * nc + lax.axis_index("c")
        base = wid * bw
        pltpu.sync_copy(idxs.at[:, pl.ds(base, bw)], idx_v)

        iota = lax.iota(jnp.int32, nl)

        def start_gather(c):
            return pltpu.async_copy(
                table.at[idx_v.at[c]], rows_v.at[lax.rem(c, _NBUF)], gsem
            )

        start_gather(0)

        def loop_body(c, carry):
            cm = lax.rem(c, _NBUF)

            @pl.when(c + 1 < s)
            def _():
                start_gather(c + 1)

            pltpu.make_async_copy(
                table.at[idx_v.at[c]], rows_v.at[cm], gsem
            ).wait()

            @pl.when(c >= _NBUF)
            def _():
                pltpu.make_async_copy(
                    trans_v.at[cm, :, pl.ds(0, bw)],
                    out.at[c - _NBUF, :, pl.ds(base, bw)],
                    wsem,
                ).wait()

            rows = rows_v.at[cm]
            trans = trans_v.at[cm]

            row_ids = [iota + h * nl for h in range(dim // nl)]
            for t in range(bw):
                col = jnp.full((nl,), t, jnp.int32)
                for h in range(dim // nl):
                    v = rows[t, pl.ds(h * nl, nl)]
                    plsc.store_scatter(trans, [row_ids[h], col], v)

            pltpu.async_copy(
                trans_v.at[cm, :, pl.ds(0, bw)],
                out.at[c, :, pl.ds(base, bw)],
                wsem,
            )
            return carry

        lax.fori_loop(0, s, loop_body, 0)
        for k in range(_NBUF):
            c = s - _NBUF + k
            pltpu.make_async_copy(
                trans_v.at[c % _NBUF, :, pl.ds(0, bw)],
                out.at[c, :, pl.ds(base, bw)],
                wsem,
            ).wait()

    return gather_kernel


def kernel(token_ids, embeddings):
    b, s = token_ids.shape
    v, d = embeddings.shape
    gather_kernel = _build(v, d, b, s)
    out_sdb = gather_kernel(embeddings, token_ids.T)
    return jnp.transpose(out_sdb, (2, 0, 1))


# speedup vs baseline: 3.8369x; 1.0003x over previous
import functools

import jax
import jax.numpy as jnp
from jax import lax
from jax.experimental import pallas as pl
from jax.experimental.pallas import tpu as pltpu
from jax.experimental.pallas import tpu_sc as plsc

_NBUF = 3


@functools.lru_cache(maxsize=None)
def _build(num_rows, dim, b, s):
    info = plsc.get_sparse_core_info()
    nc, ns, nl = info.num_cores, info.num_subcores, info.num_lanes
    nw = nc * ns
    bw = b // nw
    assert b % nw == 0 and bw % nl == 0 and dim % nl == 0

    mesh = plsc.VectorSubcoreMesh(core_axis_name="c", subcore_axis_name="s")

    @functools.partial(
        pl.kernel,
        mesh=mesh,
        compiler_params=pltpu.CompilerParams(
            use_tc_tiling_on_sc=False, needs_layout_passes=False
        ),
        out_type=jax.ShapeDtypeStruct((s, dim, b), jnp.float32),
        scratch_types=[
            pltpu.VMEM((s, bw), jnp.int32),
            pltpu.VMEM((_NBUF, bw, dim), jnp.float32),
            pltpu.VMEM((_NBUF, dim, bw + 1), jnp.float32),
            pltpu.SemaphoreType.DMA,
            pltpu.SemaphoreType.DMA,
        ],
    )
    def gather_kernel(table, idxs, out, idx_v, rows_v, trans_v, gsem, wsem):
        wid = lax.axis_index("s") * nc + lax.axis_index("c")
        base = wid * bw
        pltpu.sync_copy(idxs.at[:, pl.ds(base, bw)], idx_v)

        iota = lax.iota(jnp.int32, nl)

        def start_gather(c):
            return pltpu.async_copy(
                table.at[idx_v.at[c]], rows_v.at[lax.rem(c, _NBUF)], gsem
            )

        start_gather(0)

        def loop_body(c, carry):
            cm = lax.rem(c, _NBUF)

            @pl.when(c + 1 < s)
            def _():
                start_gather(c + 1)

            pltpu.make_async_copy(
                table.at[idx_v.at[c]], rows_v.at[cm], gsem
            ).wait()

            @pl.when(c >= _NBUF)
            def _():
                pltpu.make_async_copy(
                    trans_v.at[cm, :, pl.ds(0, bw)],
                    out.at[c - _NBUF, :, pl.ds(base, bw)],
                    wsem,
                ).wait()

            rows = rows_v.at[cm]
            trans = trans_v.at[cm]

            row_ids = [iota + h * nl for h in range(dim // nl)]
            ones = jnp.full((nl,), 1, jnp.int32)
            col = jnp.full((nl,), 0, jnp.int32)
            for t in range(bw):
                for h in range(dim // nl):
                    v = rows[t, pl.ds(h * nl, nl)]
                    plsc.store_scatter(trans, [row_ids[h], col], v)
                col = col + ones

            pltpu.async_copy(
                trans_v.at[cm, :, pl.ds(0, bw)],
                out.at[c, :, pl.ds(base, bw)],
                wsem,
            )
            return carry

        lax.fori_loop(0, s, loop_body, 0)
        for k in range(_NBUF):
            c = s - _NBUF + k
            pltpu.make_async_copy(
                trans_v.at[c % _NBUF, :, pl.ds(0, bw)],
                out.at[c, :, pl.ds(base, bw)],
                wsem,
            ).wait()

    return gather_kernel


def kernel(token_ids, embeddings):
    b, s = token_ids.shape
    v, d = embeddings.shape
    gather_kernel = _build(v, d, b, s)
    out_sdb = gather_kernel(embeddings, token_ids.T)
    return jnp.transpose(out_sdb, (2, 0, 1))
